# baseline (speedup 1.0000x reference)
import functools
import math

import jax
import jax.numpy as jnp
import numpy as np
from jax import lax
from jax.experimental import pallas as pl
from jax.experimental.pallas import tpu as pltpu
from jax.experimental.pallas import tpu_sc as plsc

_B = 2
_S = 4096
_D = 1024
_H = 16
_HD = _D // _H
_BLK = 32
_NBR = _S // _BLK
_NSEL = 512
_MB = _B * _H
_SCALE = 1.0 / math.sqrt(_HD)
_TS = 512
_G = 4
_NCH = _NSEL // _G
_DENOM = np.float32(np.float32(32.0) + np.float32(1e-6))
_NEG = -1e9


def _proj_body(x_ref, wq_ref, wk_ref, wv_ref, bq_ref, bk_ref, bv_ref,
               q_ref, k_ref, v_ref, qh_ref, kh_ref, vh_ref):
    x = x_ref[0]
    tsb = _TS // _BLK
    rb = jax.lax.broadcasted_iota(jnp.int32, (tsb, _TS), 0)
    tk = jax.lax.broadcasted_iota(jnp.int32, (tsb, _TS), 1)
    summ = (tk // _BLK == rb).astype(jnp.float32)
    for w_ref, b_ref, o_ref, h_ref in (
        (wq_ref, bq_ref, q_ref, qh_ref),
        (wk_ref, bk_ref, k_ref, kh_ref),
        (wv_ref, bv_ref, v_ref, vh_ref),
    ):
        y = jnp.dot(x, w_ref[...], preferred_element_type=jnp.float32) + b_ref[0]
        hs = jnp.dot(summ, y, preferred_element_type=jnp.float32) / _DENOM
        for hh in range(_H):
            o_ref[0, hh] = y[:, hh * _HD:(hh + 1) * _HD]
            h_ref[0, hh] = hs[:, hh * _HD:(hh + 1) * _HD]


_NV = _NBR * _NBR
_L = 16
_NGRP = _NV // _L


def _topk_sc_body(key_hbm, idx_hbm, thr_hbm, keys, hist, outidx, thrv,
                  sem):
    c = lax.axis_index("c")
    s = lax.axis_index("s")
    w = s * 2 + c

    pltpu.sync_copy(key_hbm.at[w], keys)

    lanes = lax.iota(jnp.int32, _L)

    prefix = jnp.uint32(0)
    rem = jnp.int32(_NSEL)

    for p in range(4):
        shift = 24 - 8 * p

        def hz(i, _):
            hist[pl.ds(i * _L, _L)] = jnp.zeros((_L,), jnp.int32)
            return 0

        lax.fori_loop(0, 256, hz, 0)

        pref_splat = jnp.full((_L,), prefix, dtype=jnp.uint32)

        def hacc(i, _):
            key = keys[pl.ds(i * _L, _L)]
            binv = ((key >> jnp.uint32(shift))
                    & jnp.uint32(0xFF)).astype(jnp.int32)
            addr = binv * _L + lanes
            if p == 0:
                msk = jnp.full((_L,), True)
            else:
                msk = (key >> jnp.uint32(shift + 8)) == (
                    pref_splat >> jnp.uint32(shift + 8))
            plsc.addupdate_scatter(hist, [addr], jnp.ones((_L,), jnp.int32),
                                   mask=msk)
            return 0

        lax.fori_loop(0, _NGRP, hacc, 0)

        def scan(i, st):
            cum, chosen, remin = st
            bno = 255 - i
            hv = hist[pl.ds(bno * _L, _L)]
            cnt = jnp.sum(hv)
            newcum = cum + cnt
            hit = jnp.logical_and(cum < rem, newcum >= rem)
            chosen = jnp.where(hit, bno, chosen)
            remin = jnp.where(hit, rem - cum, remin)
            return (newcum, chosen, remin)

        _, chosen, remin = lax.fori_loop(
            0, 256, scan, (jnp.int32(0), jnp.int32(0), jnp.int32(1)))
        prefix = prefix | (chosen.astype(jnp.uint32) << jnp.uint32(shift))
        rem = remin

    thr_key = prefix
    thr_splat = jnp.full((_L,), thr_key, dtype=jnp.uint32)

    def comp(i, offv):
        key = keys[pl.ds(i * _L, _L)]
        msk = key >= thr_splat
        ones = jnp.where(msk, jnp.int32(1), jnp.int32(0))
        pc = plsc.cumsum(ones)
        dest = offv + pc - 1
        okm = jnp.logical_and(msk, dest < _NSEL)
        flat = lanes + i * _L
        plsc.store_scatter(outidx, [dest], flat, mask=okm)
        cnt = plsc.all_reduce_population_count(msk)
        return offv + cnt

    lax.fori_loop(0, _NGRP, comp, jnp.zeros((_L,), jnp.int32))

    thrv[...] = jnp.full((_L,), thr_key, dtype=jnp.uint32)

    pltpu.sync_copy(outidx, idx_hbm.at[w])
    pltpu.sync_copy(thrv, thr_hbm.at[w])


def _topk_sc(lnorm_flat):
    mesh = plsc.VectorSubcoreMesh(core_axis_name="c", subcore_axis_name="s")
    k = functools.partial(
        pl.kernel,
        mesh=mesh,
        compiler_params=pltpu.CompilerParams(needs_layout_passes=False),
        out_type=[
            jax.ShapeDtypeStruct((_MB, _NSEL), jnp.int32),
            jax.ShapeDtypeStruct((_MB, _L), jnp.uint32),
        ],
        scratch_types=[
            pltpu.VMEM((_NV,), jnp.uint32),
            pltpu.VMEM((256 * _L,), jnp.int32),
            pltpu.VMEM((_NSEL,), jnp.int32),
            pltpu.VMEM((_L,), jnp.uint32),
            pltpu.SemaphoreType.DMA,
        ],
    )(_topk_sc_body)
    return k(lnorm_flat)


def _lowlogit_body(qh_ref, kh_ref, ll_ref, key_ref):
    ll = jax.lax.dot_general(
        qh_ref[0, 0], kh_ref[0, 0], (((1,), (1,)), ((), ())),
        preferred_element_type=jnp.float32) * _SCALE
    ll_ref[0, 0] = ll
    lnorm = ll - jnp.max(ll, axis=1, keepdims=True)
    u = jax.lax.bitcast_convert_type(lnorm, jnp.uint32)
    neg = u >= jnp.uint32(0x80000000)
    key_ref[0, 0] = jnp.where(neg,
                              u ^ jnp.uint32(0xFFFFFFFF),
                              u | jnp.uint32(0x80000000))


def _attn_body(rows_sm, cols_sm, thr_sm,
               q_ref, k_ref, v_ref, vh_ref, ll_ref,
               out_ref, mx, hn, acc):
    b = pl.program_id(0)
    h2 = pl.program_id(1)
    gb = _G * _BLK

    ri = jax.lax.broadcasted_iota(jnp.int32, (gb, gb), 0) // _BLK
    ci = jax.lax.broadcasted_iota(jnp.int32, (gb, gb), 1) // _BLK
    bd = ri == ci

    for hh in range(2):
        bh = b * _H + h2 * 2 + hh
        acc[...] = jnp.zeros_like(acc)
        hn[...] = jnp.zeros_like(hn)
        mx[...] = jnp.full_like(mx, _NEG)

        def gather3(base):
            qs = jnp.concatenate(
                [q_ref[0, hh, pl.ds(rows_sm[bh, base + j] * _BLK, _BLK), :]
                 for j in range(_G)], axis=0)
            ks = jnp.concatenate(
                [k_ref[0, hh, pl.ds(cols_sm[bh, base + j] * _BLK, _BLK), :]
                 for j in range(_G)], axis=0)
            return qs, ks

        def logits(qs, ks):
            return jax.lax.dot_general(
                qs, ks, (((1,), (1,)), ((), ())),
                preferred_element_type=jnp.float32) * _SCALE

        def onepass(n, carry):
            base = n * _G
            qs, ks = gather3(base)
            vs = jnp.concatenate(
                [v_ref[0, hh, pl.ds(cols_sm[bh, base + j] * _BLK, _BLK), :]
                 for j in range(_G)], axis=0)
            lg = jnp.where(bd, logits(qs, ks), _NEG)
            m = jnp.max(lg, axis=1, keepdims=True)
            news = []
            scales = []
            for j in range(_G):
                sl = pl.ds(rows_sm[bh, base + j] * _BLK, _BLK)
                js = slice(j * _BLK, (j + 1) * _BLK)
                mold = mx[sl, :]
                mnew = jnp.maximum(mold, m[js, :])
                mx[sl, :] = mnew
                news.append(mnew)
                scales.append(jnp.exp(mold - mnew))
            mv = jnp.concatenate(news, axis=0)
            p = jnp.exp(lg - mv)
            po = jnp.dot(p, vs, preferred_element_type=jnp.float32)
            pn = jnp.sum(p, axis=1, keepdims=True)
            for j in range(_G):
                sl = pl.ds(rows_sm[bh, base + j] * _BLK, _BLK)
                js = slice(j * _BLK, (j + 1) * _BLK)
                acc[sl, :] = acc[sl, :] * scales[j] + po[js, :]
                hn[sl, :] = hn[sl, :] * scales[j] + pn[js, :]
            return carry

        jax.lax.fori_loop(0, _NCH, onepass, 0, unroll=8)

        ll = ll_ref[0, hh]
        rmax = jnp.max(ll, axis=1, keepdims=True)
        lnorm = ll - rmax
        thr = thr_sm[bh]
        la = jnp.exp(jnp.where(lnorm >= thr, lnorm - 10000.0, lnorm)) * 32.0
        lo = jnp.dot(la, vh_ref[0, hh], preferred_element_type=jnp.float32)
        ln = jnp.sum(la, axis=1, keepdims=True)
        lrn = jnp.concatenate([lo, ln, rmax], axis=1)

        lt = jnp.broadcast_to(lrn[:, None, :], (_NBR, _BLK, _HD + 2)).reshape(
            _S, _HD + 2)
        lo_t = lt[:, :_HD]
        ln_t = lt[:, _HD:_HD + 1]
        rm_t = lt[:, _HD + 1:_HD + 2]
        logc = rm_t - mx[...]
        lcorr = jnp.exp(jnp.where(logc <= 0.0, logc, 0.0))
        hcorr = jnp.exp(jnp.where(logc > 0.0, -logc, 0.0))
        num = acc[...] * hcorr + lo_t * lcorr
        den = hn[...] * hcorr + ln_t * lcorr + 1e-6
        out_ref[0, :, hh * _HD:(hh + 1) * _HD] = num / den


def _build_calls(interpret=False):
    proj = pl.pallas_call(
        _proj_body,
        grid=(_B, _S // _TS),
        in_specs=[
            pl.BlockSpec((1, _TS, _D), lambda b, t: (b, t, 0)),
            pl.BlockSpec((_D, _D), lambda b, t: (0, 0)),
            pl.BlockSpec((_D, _D), lambda b, t: (0, 0)),
            pl.BlockSpec((_D, _D), lambda b, t: (0, 0)),
            pl.BlockSpec((1, _D), lambda b, t: (0, 0)),
            pl.BlockSpec((1, _D), lambda b, t: (0, 0)),
            pl.BlockSpec((1, _D), lambda b, t: (0, 0)),
        ],
        out_specs=[
            pl.BlockSpec((1, _H, _TS, _HD), lambda b, t: (b, 0, t, 0)),
            pl.BlockSpec((1, _H, _TS, _HD), lambda b, t: (b, 0, t, 0)),
            pl.BlockSpec((1, _H, _TS, _HD), lambda b, t: (b, 0, t, 0)),
            pl.BlockSpec((1, _H, _TS // _BLK, _HD), lambda b, t: (b, 0, t, 0)),
            pl.BlockSpec((1, _H, _TS // _BLK, _HD), lambda b, t: (b, 0, t, 0)),
            pl.BlockSpec((1, _H, _TS // _BLK, _HD), lambda b, t: (b, 0, t, 0)),
        ],
        out_shape=[
            jax.ShapeDtypeStruct((_B, _H, _S, _HD), jnp.float32),
            jax.ShapeDtypeStruct((_B, _H, _S, _HD), jnp.float32),
            jax.ShapeDtypeStruct((_B, _H, _S, _HD), jnp.float32),
            jax.ShapeDtypeStruct((_B, _H, _NBR, _HD), jnp.float32),
            jax.ShapeDtypeStruct((_B, _H, _NBR, _HD), jnp.float32),
            jax.ShapeDtypeStruct((_B, _H, _NBR, _HD), jnp.float32),
        ],
        interpret=interpret,
    )

    lowlogit = pl.pallas_call(
        _lowlogit_body,
        grid=(_B, _H),
        in_specs=[
            pl.BlockSpec((1, 1, _NBR, _HD), lambda b, h: (b, h, 0, 0)),
            pl.BlockSpec((1, 1, _NBR, _HD), lambda b, h: (b, h, 0, 0)),
        ],
        out_specs=[
            pl.BlockSpec((1, 1, _NBR, _NBR), lambda b, h: (b, h, 0, 0)),
            pl.BlockSpec((1, 1, _NBR, _NBR), lambda b, h: (b, h, 0, 0)),
        ],
        out_shape=[
            jax.ShapeDtypeStruct((_B, _H, _NBR, _NBR), jnp.float32),
            jax.ShapeDtypeStruct((_B, _H, _NBR, _NBR), jnp.uint32),
        ],
        interpret=interpret,
    )

    attn = pl.pallas_call(
        _attn_body,
        grid_spec=pltpu.PrefetchScalarGridSpec(
            num_scalar_prefetch=3,
            grid=(_B, _H // 2),
            in_specs=[
                pl.BlockSpec((1, 2, _S, _HD), lambda b, h, *_: (b, h, 0, 0)),
                pl.BlockSpec((1, 2, _S, _HD), lambda b, h, *_: (b, h, 0, 0)),
                pl.BlockSpec((1, 2, _S, _HD), lambda b, h, *_: (b, h, 0, 0)),
                pl.BlockSpec((1, 2, _NBR, _HD), lambda b, h, *_: (b, h, 0, 0)),
                pl.BlockSpec((1, 2, _NBR, _NBR), lambda b, h, *_: (b, h, 0, 0)),
            ],
            out_specs=pl.BlockSpec((1, _S, 2 * _HD), lambda b, h, *_: (b, 0, h)),
            scratch_shapes=[
                pltpu.VMEM((_S, 1), jnp.float32),
                pltpu.VMEM((_S, 1), jnp.float32),
                pltpu.VMEM((_S, _HD), jnp.float32),
            ],
        ),
        out_shape=jax.ShapeDtypeStruct((_B, _S, _D), jnp.float32),
        interpret=interpret,
    )
    return proj, lowlogit, attn


def _run(hidden_states, attention_mask, Wq, bq, Wk, bk, Wv, bv, interpret=False):
    proj, lowlogit, attn = _build_calls(interpret)
    q, k, v, qh, kh, vh = proj(
        hidden_states, Wq, Wk, Wv,
        bq.reshape(1, _D), bk.reshape(1, _D), bv.reshape(1, _D))
    ll, keys = lowlogit(qh, kh)
    flat = keys.reshape(_MB, _NBR * _NBR)
    top_idx, thrm = _topk_sc(flat)
    tk = thrm[:, 0]
    tu = jnp.where(tk < jnp.uint32(0x80000000),
                   tk ^ jnp.uint32(0xFFFFFFFF),
                   tk & jnp.uint32(0x7FFFFFFF))
    thr = jax.lax.bitcast_convert_type(tu, jnp.float32)
    rows = top_idx // _NBR
    cols = top_idx % _NBR
    return attn(rows, cols, thr, q, k, v, vh, ll)


def kernel(hidden_states, attention_mask, Wq, bq, Wk, bk, Wv, bv):
    return _run(hidden_states, attention_mask, Wq, bq, Wk, bk, Wv, bv,
                interpret=False)


# speedup vs baseline: 1.3060x; 1.3060x over previous
import functools
import math

import jax
import jax.numpy as jnp
import numpy as np
from jax import lax
from jax.experimental import pallas as pl
from jax.experimental.pallas import tpu as pltpu
from jax.experimental.pallas import tpu_sc as plsc

_B = 2
_S = 4096
_D = 1024
_H = 16
_HD = _D // _H
_BLK = 32
_NBR = _S // _BLK
_NSEL = 512
_MB = _B * _H
_SCALE = 1.0 / math.sqrt(_HD)
_TS = 512
_G = 4
_NCH = _NSEL // _G
_DENOM = np.float32(np.float32(32.0) + np.float32(1e-6))
_NEG = -1e9


def _proj_body(x_ref, wq_ref, wk_ref, wv_ref, bq_ref, bk_ref, bv_ref,
               q_ref, k_ref, v_ref, qh_ref, kh_ref, vh_ref):
    x = x_ref[0]
    tsb = _TS // _BLK
    rb = jax.lax.broadcasted_iota(jnp.int32, (tsb, _TS), 0)
    tk = jax.lax.broadcasted_iota(jnp.int32, (tsb, _TS), 1)
    summ = (tk // _BLK == rb).astype(jnp.float32)
    for w_ref, b_ref, o_ref, h_ref in (
        (wq_ref, bq_ref, q_ref, qh_ref),
        (wk_ref, bk_ref, k_ref, kh_ref),
        (wv_ref, bv_ref, v_ref, vh_ref),
    ):
        y = jnp.dot(x, w_ref[...], preferred_element_type=jnp.float32) + b_ref[0]
        hs = jnp.dot(summ, y, preferred_element_type=jnp.float32) / _DENOM
        for hh in range(_H):
            o_ref[0, hh] = y[:, hh * _HD:(hh + 1) * _HD]
            h_ref[0, hh] = hs[:, hh * _HD:(hh + 1) * _HD]


_NV = _NBR * _NBR
_L = 16
_NGRP = _NV // _L


def _topk_sc_body(key_hbm, idx_hbm, thr_hbm, keys, hist, outidx, thrv,
                  sem):
    c = lax.axis_index("c")
    s = lax.axis_index("s")
    w = s * 2 + c

    pltpu.sync_copy(key_hbm.at[w], keys)

    lanes = lax.iota(jnp.int32, _L)

    prefix = jnp.uint32(0)
    rem = jnp.int32(_NSEL)

    for p in range(4):
        shift = 24 - 8 * p

        def hz(i, _):
            hist[pl.ds(i * _L, _L)] = jnp.zeros((_L,), jnp.int32)
            return 0

        lax.fori_loop(0, 256, hz, 0)

        pref_splat = jnp.full((_L,), prefix, dtype=jnp.uint32)

        def hacc(i, _):
            key = keys[pl.ds(i * _L, _L)]
            binv = ((key >> jnp.uint32(shift))
                    & jnp.uint32(0xFF)).astype(jnp.int32)
            addr = binv * _L + lanes
            if p == 0:
                msk = jnp.full((_L,), True)
            else:
                msk = (key >> jnp.uint32(shift + 8)) == (
                    pref_splat >> jnp.uint32(shift + 8))
            plsc.addupdate_scatter(hist, [addr], jnp.ones((_L,), jnp.int32),
                                   mask=msk)
            return 0

        lax.fori_loop(0, _NGRP, hacc, 0)

        def scan(i, st):
            cum, chosen, remin = st
            bno = 255 - i
            hv = hist[pl.ds(bno * _L, _L)]
            cnt = jnp.sum(hv)
            newcum = cum + cnt
            hit = jnp.logical_and(cum < rem, newcum >= rem)
            chosen = jnp.where(hit, bno, chosen)
            remin = jnp.where(hit, rem - cum, remin)
            return (newcum, chosen, remin)

        _, chosen, remin = lax.fori_loop(
            0, 256, scan, (jnp.int32(0), jnp.int32(0), jnp.int32(1)))
        prefix = prefix | (chosen.astype(jnp.uint32) << jnp.uint32(shift))
        rem = remin

    thr_key = prefix
    thr_splat = jnp.full((_L,), thr_key, dtype=jnp.uint32)

    def comp(i, offv):
        key = keys[pl.ds(i * _L, _L)]
        msk = key >= thr_splat
        ones = jnp.where(msk, jnp.int32(1), jnp.int32(0))
        pc = plsc.cumsum(ones)
        dest = offv + pc - 1
        okm = jnp.logical_and(msk, dest < _NSEL)
        flat = lanes + i * _L
        plsc.store_scatter(outidx, [dest], flat, mask=okm)
        cnt = plsc.all_reduce_population_count(msk)
        return offv + cnt

    lax.fori_loop(0, _NGRP, comp, jnp.zeros((_L,), jnp.int32))

    thrv[...] = jnp.full((_L,), thr_key, dtype=jnp.uint32)

    pltpu.sync_copy(outidx, idx_hbm.at[w])
    pltpu.sync_copy(thrv, thr_hbm.at[w])


def _topk_sc(lnorm_flat):
    mesh = plsc.VectorSubcoreMesh(core_axis_name="c", subcore_axis_name="s")
    k = functools.partial(
        pl.kernel,
        mesh=mesh,
        compiler_params=pltpu.CompilerParams(needs_layout_passes=False),
        out_type=[
            jax.ShapeDtypeStruct((_MB, _NSEL), jnp.int32),
            jax.ShapeDtypeStruct((_MB, _L), jnp.uint32),
        ],
        scratch_types=[
            pltpu.VMEM((_NV,), jnp.uint32),
            pltpu.VMEM((256 * _L,), jnp.int32),
            pltpu.VMEM((_NSEL,), jnp.int32),
            pltpu.VMEM((_L,), jnp.uint32),
            pltpu.SemaphoreType.DMA,
        ],
    )(_topk_sc_body)
    return k(lnorm_flat)


def _lowlogit_body(qh_ref, kh_ref, ll_ref, key_ref):
    ll = jax.lax.dot_general(
        qh_ref[0, 0], kh_ref[0, 0], (((1,), (1,)), ((), ())),
        preferred_element_type=jnp.float32) * _SCALE
    ll_ref[0, 0] = ll
    lnorm = ll - jnp.max(ll, axis=1, keepdims=True)
    u = jax.lax.bitcast_convert_type(lnorm, jnp.uint32)
    neg = u >= jnp.uint32(0x80000000)
    key_ref[0, 0] = jnp.where(neg,
                              u ^ jnp.uint32(0xFFFFFFFF),
                              u | jnp.uint32(0x80000000))


def _attn_body(rows_sm, cols_sm, thr_sm,
               q_ref, k_ref, v_ref, vh_ref, ll_ref,
               out_ref, mx, hn, acc):
    b = pl.program_id(0)
    h2 = pl.program_id(1)
    gb = _G * _BLK

    ri = jax.lax.broadcasted_iota(jnp.int32, (gb, gb), 0) // _BLK
    ci = jax.lax.broadcasted_iota(jnp.int32, (gb, gb), 1) // _BLK
    bd = ri == ci

    bhs = [b * _H + h2 * 2 + hh for hh in range(2)]

    acc[...] = jnp.zeros_like(acc)
    hn[...] = jnp.zeros_like(hn)
    mx[...] = jnp.full_like(mx, _NEG)

    def gather2(hh, bh, base):
        qs = jnp.concatenate(
            [q_ref[0, hh, pl.ds(rows_sm[bh, base + j] * _BLK, _BLK), :]
             for j in range(_G)], axis=0)
        ks = jnp.concatenate(
            [k_ref[0, hh, pl.ds(cols_sm[bh, base + j] * _BLK, _BLK), :]
             for j in range(_G)], axis=0)
        return qs, ks

    def logits(qs, ks):
        return jax.lax.dot_general(
            qs, ks, (((1,), (1,)), ((), ())),
            preferred_element_type=jnp.float32) * _SCALE

    def pass1(n, carry):
        base = n * _G
        for hh in range(2):
            bh = bhs[hh]
            qs, ks = gather2(hh, bh, base)
            lg = jnp.where(bd, logits(qs, ks), _NEG)
            m = jnp.max(lg, axis=1, keepdims=True)
            for j in range(_G):
                sl = pl.ds(rows_sm[bh, base + j] * _BLK, _BLK)
                mx[hh, sl, :] = jnp.maximum(mx[hh, sl, :],
                                            m[j * _BLK:(j + 1) * _BLK, :])
        return carry

    jax.lax.fori_loop(0, _NCH, pass1, 0, unroll=4)

    def pass2(n, carry):
        base = n * _G
        for hh in range(2):
            bh = bhs[hh]
            qs, ks = gather2(hh, bh, base)
            vs = jnp.concatenate(
                [v_ref[0, hh, pl.ds(cols_sm[bh, base + j] * _BLK, _BLK), :]
                 for j in range(_G)], axis=0)
            mv = jnp.concatenate(
                [mx[hh, pl.ds(rows_sm[bh, base + j] * _BLK, _BLK), :]
                 for j in range(_G)], axis=0)
            lg = jnp.where(bd, logits(qs, ks) - mv, _NEG)
            p = jnp.exp(lg)
            po = jnp.dot(p, vs, preferred_element_type=jnp.float32)
            pn = jnp.sum(p, axis=1, keepdims=True)
            for j in range(_G):
                sl = pl.ds(rows_sm[bh, base + j] * _BLK, _BLK)
                js = slice(j * _BLK, (j + 1) * _BLK)
                acc[hh, sl, :] += po[js, :]
                hn[hh, sl, :] += pn[js, :]
        return carry

    jax.lax.fori_loop(0, _NCH, pass2, 0, unroll=4)

    for hh in range(2):
        bh = bhs[hh]
        ll = ll_ref[0, hh]
        rmax = jnp.max(ll, axis=1, keepdims=True)
        lnorm = ll - rmax
        thr = thr_sm[bh]
        la = jnp.exp(jnp.where(lnorm >= thr, lnorm - 10000.0, lnorm)) * 32.0
        lo = jnp.dot(la, vh_ref[0, hh], preferred_element_type=jnp.float32)
        ln = jnp.sum(la, axis=1, keepdims=True)
        lrn = jnp.concatenate([lo, ln, rmax], axis=1)

        lt = jnp.broadcast_to(lrn[:, None, :], (_NBR, _BLK, _HD + 2)).reshape(
            _S, _HD + 2)
        lo_t = lt[:, :_HD]
        ln_t = lt[:, _HD:_HD + 1]
        rm_t = lt[:, _HD + 1:_HD + 2]
        logc = rm_t - mx[hh]
        lcorr = jnp.exp(jnp.where(logc <= 0.0, logc, 0.0))
        hcorr = jnp.exp(jnp.where(logc > 0.0, -logc, 0.0))
        num = acc[hh] * hcorr + lo_t * lcorr
        den = hn[hh] * hcorr + ln_t * lcorr + 1e-6
        out_ref[0, :, hh * _HD:(hh + 1) * _HD] = num / den


def _build_calls(interpret=False):
    proj = pl.pallas_call(
        _proj_body,
        grid=(_B, _S // _TS),
        in_specs=[
            pl.BlockSpec((1, _TS, _D), lambda b, t: (b, t, 0)),
            pl.BlockSpec((_D, _D), lambda b, t: (0, 0)),
            pl.BlockSpec((_D, _D), lambda b, t: (0, 0)),
            pl.BlockSpec((_D, _D), lambda b, t: (0, 0)),
            pl.BlockSpec((1, _D), lambda b, t: (0, 0)),
            pl.BlockSpec((1, _D), lambda b, t: (0, 0)),
            pl.BlockSpec((1, _D), lambda b, t: (0, 0)),
        ],
        out_specs=[
            pl.BlockSpec((1, _H, _TS, _HD), lambda b, t: (b, 0, t, 0)),
            pl.BlockSpec((1, _H, _TS, _HD), lambda b, t: (b, 0, t, 0)),
            pl.BlockSpec((1, _H, _TS, _HD), lambda b, t: (b, 0, t, 0)),
            pl.BlockSpec((1, _H, _TS // _BLK, _HD), lambda b, t: (b, 0, t, 0)),
            pl.BlockSpec((1, _H, _TS // _BLK, _HD), lambda b, t: (b, 0, t, 0)),
            pl.BlockSpec((1, _H, _TS // _BLK, _HD), lambda b, t: (b, 0, t, 0)),
        ],
        out_shape=[
            jax.ShapeDtypeStruct((_B, _H, _S, _HD), jnp.float32),
            jax.ShapeDtypeStruct((_B, _H, _S, _HD), jnp.float32),
            jax.ShapeDtypeStruct((_B, _H, _S, _HD), jnp.float32),
            jax.ShapeDtypeStruct((_B, _H, _NBR, _HD), jnp.float32),
            jax.ShapeDtypeStruct((_B, _H, _NBR, _HD), jnp.float32),
            jax.ShapeDtypeStruct((_B, _H, _NBR, _HD), jnp.float32),
        ],
        interpret=interpret,
    )

    lowlogit = pl.pallas_call(
        _lowlogit_body,
        grid=(_B, _H),
        in_specs=[
            pl.BlockSpec((1, 1, _NBR, _HD), lambda b, h: (b, h, 0, 0)),
            pl.BlockSpec((1, 1, _NBR, _HD), lambda b, h: (b, h, 0, 0)),
        ],
        out_specs=[
            pl.BlockSpec((1, 1, _NBR, _NBR), lambda b, h: (b, h, 0, 0)),
            pl.BlockSpec((1, 1, _NBR, _NBR), lambda b, h: (b, h, 0, 0)),
        ],
        out_shape=[
            jax.ShapeDtypeStruct((_B, _H, _NBR, _NBR), jnp.float32),
            jax.ShapeDtypeStruct((_B, _H, _NBR, _NBR), jnp.uint32),
        ],
        interpret=interpret,
    )

    attn = pl.pallas_call(
        _attn_body,
        grid_spec=pltpu.PrefetchScalarGridSpec(
            num_scalar_prefetch=3,
            grid=(_B, _H // 2),
            in_specs=[
                pl.BlockSpec((1, 2, _S, _HD), lambda b, h, *_: (b, h, 0, 0)),
                pl.BlockSpec((1, 2, _S, _HD), lambda b, h, *_: (b, h, 0, 0)),
                pl.BlockSpec((1, 2, _S, _HD), lambda b, h, *_: (b, h, 0, 0)),
                pl.BlockSpec((1, 2, _NBR, _HD), lambda b, h, *_: (b, h, 0, 0)),
                pl.BlockSpec((1, 2, _NBR, _NBR), lambda b, h, *_: (b, h, 0, 0)),
            ],
            out_specs=pl.BlockSpec((1, _S, 2 * _HD), lambda b, h, *_: (b, 0, h)),
            scratch_shapes=[
                pltpu.VMEM((2, _S, 1), jnp.float32),
                pltpu.VMEM((2, _S, 1), jnp.float32),
                pltpu.VMEM((2, _S, _HD), jnp.float32),
            ],
        ),
        out_shape=jax.ShapeDtypeStruct((_B, _S, _D), jnp.float32),
        interpret=interpret,
    )
    return proj, lowlogit, attn


def _run(hidden_states, attention_mask, Wq, bq, Wk, bk, Wv, bv, interpret=False):
    proj, lowlogit, attn = _build_calls(interpret)
    q, k, v, qh, kh, vh = proj(
        hidden_states, Wq, Wk, Wv,
        bq.reshape(1, _D), bk.reshape(1, _D), bv.reshape(1, _D))
    ll, keys = lowlogit(qh, kh)
    flat = keys.reshape(_MB, _NBR * _NBR)
    top_idx, thrm = _topk_sc(flat)
    tk = thrm[:, 0]
    tu = jnp.where(tk < jnp.uint32(0x80000000),
                   tk ^ jnp.uint32(0xFFFFFFFF),
                   tk & jnp.uint32(0x7FFFFFFF))
    thr = jax.lax.bitcast_convert_type(tu, jnp.float32)
    rows = top_idx // _NBR
    cols = top_idx % _NBR
    return attn(rows, cols, thr, q, k, v, vh, ll)


def kernel(hidden_states, attention_mask, Wq, bq, Wk, bk, Wv, bv):
    return _run(hidden_states, attention_mask, Wq, bq, Wk, bk, Wv, bv,
                interpret=False)
